# baseline (speedup 1.0000x reference)
import functools

import jax
import jax.numpy as jnp
from jax import lax
from jax.experimental import pallas as pl
from jax.experimental.pallas import tpu as pltpu
from jax.experimental.pallas import tpu_sc as plsc

NP = 10240
HD = 128
NG = 64
CH = 128
NCH = 80
TILES = 32
EPAD = TILES * NCH * CH
DUMMY = 10000
RPT = NP // 16
BR = 1024
GRID = NP // BR

CHA = 64
NCHF = 240
NCHS = 76
NGF = NCHF // 2
NGS = NCHS // 2
EDGF = 16 * NCHF * CHA
EPAD_A = 16 * (NCHF + NCHS) * CHA


def _agg_body(v_hbm, src_hbm, dst_hbm, z_hbm, out_hbm,
              acc, srci, dsti, rows, g0, g1, g2, g3, t0, t1, t2, t3):
    gsem = (g0, g1, g2, g3)
    ssem = (t0, t1, t2, t3)
    c = lax.axis_index("c")
    s = lax.axis_index("s")
    wid = c * 16 + s

    pltpu.sync_copy(z_hbm, acc.at[pl.ds(s * RPT, RPT)])
    plsc.subcore_barrier()

    ngroups = lax.select(c == 0, NGF, NGS)
    npairs = lax.select(c == 0, NGF // 2, NGS // 2)

    def stage(G, p):
        pltpu.sync_copy(src_hbm.at[wid, pl.ds(2 * G, 2)], srci.at[p])
        pltpu.sync_copy(dst_hbm.at[wid, pl.ds(2 * G, 2)], dsti.at[p])

    def gath(p, b):
        pltpu.async_copy(v_hbm.at[srci.at[p, b]], rows.at[2 * p + b],
                         gsem[2 * p + b])

    def wait_gath(p, b):
        pltpu.make_async_copy(v_hbm.at[srci.at[p, b]], rows.at[2 * p + b],
                              gsem[2 * p + b]).wait()

    def scat(p, b):
        pltpu.async_copy(rows.at[2 * p + b], acc.at[dsti.at[p, b]],
                         ssem[2 * p + b], add=True)

    def wait_scat(p, b):
        pltpu.make_async_copy(rows.at[2 * p + b], acc.at[dsti.at[p, b]],
                              ssem[2 * p + b]).wait()

    stage(0, 0)
    for b in range(2):
        gath(0, b)

    def pair(k, carry):
        for p in (0, 1):
            G = 2 * k + p
            pn = 1 - p
            for b in range(2):
                wait_gath(p, b)
                scat(p, b)

            @pl.when(G >= 1)
            def _drain():
                for b in range(2):
                    wait_scat(pn, b)

            @pl.when(G + 1 < ngroups)
            def _next():
                stage(G + 1, pn)
                for b in range(2):
                    gath(pn, b)
        return carry

    lax.fori_loop(0, npairs, pair, 0)

    pf = (ngroups - 1) % 2

    @pl.when(pf == 0)
    def _d0():
        for b in range(2):
            wait_scat(0, b)

    @pl.when(pf == 1)
    def _d1():
        for b in range(2):
            wait_scat(1, b)

    plsc.subcore_barrier()

    for i in range(RPT // CHA):
        r0 = s * RPT + i * CHA
        pltpu.sync_copy(acc.at[pl.ds(r0, CHA)], rows.at[0])
        pltpu.sync_copy(rows.at[0], out_hbm.at[c, pl.ds(r0, CHA)])


@functools.lru_cache(maxsize=None)
def _get_agg():
    return pl.kernel(
        _agg_body,
        out_type=jax.ShapeDtypeStruct((2, NP, HD), jnp.float32),
        mesh=plsc.VectorSubcoreMesh(core_axis_name="c", subcore_axis_name="s"),
        scratch_types=[
            pltpu.VMEM_SHARED((NP, HD), jnp.float32),
            pltpu.VMEM((2, 2, CHA), jnp.int32),
            pltpu.VMEM((2, 2, CHA), jnp.int32),
            pltpu.VMEM((4, CHA, HD), jnp.float32),
            pltpu.SemaphoreType.DMA,
            pltpu.SemaphoreType.DMA,
            pltpu.SemaphoreType.DMA,
            pltpu.SemaphoreType.DMA,
            pltpu.SemaphoreType.DMA,
            pltpu.SemaphoreType.DMA,
            pltpu.SemaphoreType.DMA,
            pltpu.SemaphoreType.DMA,
        ],
    )


def _deg_body(dst_hbm, z_hbm, ones_hbm, dout_hbm, dacc, dsti, ones_v):
    c = lax.axis_index("c")
    s = lax.axis_index("s")
    wid = c * 16 + s

    pltpu.sync_copy(z_hbm, dacc.at[pl.ds(s * RPT, RPT)])
    pltpu.sync_copy(ones_hbm, ones_v)
    plsc.subcore_barrier()

    def group(g, carry):
        pltpu.sync_copy(dst_hbm.at[wid, pl.ds(2 * g, 2)], dsti)
        for b in range(2):
            pltpu.sync_copy(ones_v, dacc.at[dsti.at[b]], add=True)
        return carry

    lax.fori_loop(0, NCH // 2, group, 0)
    plsc.subcore_barrier()

    for i in range(RPT // CH):
        r0 = s * RPT + i * CH
        pltpu.sync_copy(dacc.at[pl.ds(r0, CH)], ones_v)
        pltpu.sync_copy(ones_v, dout_hbm.at[c, pl.ds(r0, CH)])


@functools.lru_cache(maxsize=None)
def _get_deg():
    return pl.kernel(
        _deg_body,
        out_type=jax.ShapeDtypeStruct((2, NP, HD), jnp.float32),
        mesh=plsc.VectorSubcoreMesh(core_axis_name="c", subcore_axis_name="s"),
        scratch_types=[
            pltpu.VMEM_SHARED((NP, HD), jnp.float32),
            pltpu.VMEM((2, CH), jnp.int32),
            pltpu.VMEM((CH, HD), jnp.float32),
        ],
    )


def _dense_body(x_ref, w_ref, b_ref, o_ref):
    y = jnp.dot(x_ref[...], w_ref[...], preferred_element_type=jnp.float32)
    o_ref[...] = jnp.maximum(y + b_ref[...], 0.0)


_dense = pl.pallas_call(
    _dense_body,
    grid=(GRID,),
    in_specs=[
        pl.BlockSpec((BR, HD), lambda i: (i, 0)),
        pl.BlockSpec((HD, HD), lambda i: (0, 0)),
        pl.BlockSpec((1, HD), lambda i: (0, 0)),
    ],
    out_specs=pl.BlockSpec((BR, HD), lambda i: (i, 0)),
    out_shape=jax.ShapeDtypeStruct((NP, HD), jnp.float32),
)


def _combine_body(t_ref, p_ref, d_ref, wl_ref, wr_ref, b_ref, h_ref, s_ref):
    t = t_ref[...]
    agg = p_ref[0] + p_ref[1]
    deg = d_ref[0, :, 0:1] + d_ref[1, :, 0:1]
    mean = agg / jnp.maximum(deg, 1.0)
    y = jnp.dot(t, wl_ref[...], preferred_element_type=jnp.float32)
    y = y + jnp.dot(mean, wr_ref[...], preferred_element_type=jnp.float32)
    h = jnp.maximum(y + b_ref[...], 0.0)
    h_ref[...] = h
    s_ref[...] = t + h


_combine = pl.pallas_call(
    _combine_body,
    grid=(GRID,),
    in_specs=[
        pl.BlockSpec((BR, HD), lambda i: (i, 0)),
        pl.BlockSpec((2, BR, HD), lambda i: (0, i, 0)),
        pl.BlockSpec((2, BR, HD), lambda i: (0, i, 0)),
        pl.BlockSpec((HD, HD), lambda i: (0, 0)),
        pl.BlockSpec((HD, HD), lambda i: (0, 0)),
        pl.BlockSpec((1, HD), lambda i: (0, 0)),
    ],
    out_specs=[
        pl.BlockSpec((BR, HD), lambda i: (i, 0)),
        pl.BlockSpec((BR, HD), lambda i: (i, 0)),
    ],
    out_shape=[
        jax.ShapeDtypeStruct((NP, HD), jnp.float32),
        jax.ShapeDtypeStruct((NP, HD), jnp.float32),
    ],
)


def _readout_body(h_ref, bat_ref, row_ref, rob_ref, clw_ref, clb_ref,
                  o_ref, accy, accc):
    i = pl.program_id(0)
    bids = bat_ref[0, 0, :]
    onehot = (bids[:, None] ==
              lax.broadcasted_iota(jnp.int32, (BR, NG), 1)).astype(jnp.float32)
    dn = (((0,), (0,)), ((), ()))
    py = lax.dot_general(onehot, h_ref[...], dn,
                         preferred_element_type=jnp.float32)
    pc = lax.dot_general(onehot, jnp.ones((BR, HD), jnp.float32), dn,
                         preferred_element_type=jnp.float32)

    @pl.when(i == 0)
    def _init():
        accy[...] = py
        accc[...] = pc

    @pl.when(i > 0)
    def _accum():
        accy[...] = accy[...] + py
        accc[...] = accc[...] + pc

    @pl.when(i == GRID - 1)
    def _final():
        pooled = accy[...] / jnp.maximum(accc[...], 1.0)
        r = jnp.maximum(
            jnp.dot(pooled, row_ref[...], preferred_element_type=jnp.float32)
            + rob_ref[...], 0.0)
        o_ref[...] = (jnp.dot(r, clw_ref[...],
                              preferred_element_type=jnp.float32)
                      + clb_ref[...])


_readout = pl.pallas_call(
    _readout_body,
    grid=(GRID,),
    in_specs=[
        pl.BlockSpec((BR, HD), lambda i: (i, 0)),
        pl.BlockSpec((1, 1, BR), lambda i: (i, 0, 0)),
        pl.BlockSpec((HD, HD), lambda i: (0, 0)),
        pl.BlockSpec((1, HD), lambda i: (0, 0)),
        pl.BlockSpec((HD, HD), lambda i: (0, 0)),
        pl.BlockSpec((1, HD), lambda i: (0, 0)),
    ],
    out_specs=pl.BlockSpec((NG, HD), lambda i: (0, 0)),
    out_shape=jax.ShapeDtypeStruct((NG, HD), jnp.float32),
    scratch_shapes=[
        pltpu.VMEM((NG, HD), jnp.float32),
        pltpu.VMEM((NG, HD), jnp.float32),
    ],
)


def kernel(x, edge_index, batch, lin1_W, lin1_b, sage_Wl, sage_Wr, sage_b,
           cell_W, cell_b, ro_W, ro_b, cls_W, cls_b):
    n, d = x.shape
    e = edge_index.shape[1]
    out_dim = cls_W.shape[1]

    xp = jnp.pad(x, ((0, NP - n), (0, 0)))
    dst3 = jnp.pad(edge_index[1], (0, EPAD - e),
                   constant_values=DUMMY).reshape(TILES, NCH, CH)

    def asym(idx, fill):
        fast = idx[:EDGF].reshape(16, NCHF, CHA)
        slow = jnp.pad(idx[EDGF:], (0, EPAD_A - EDGF - (e - EDGF)),
                       constant_values=fill).reshape(16, NCHS, CHA)
        slow = jnp.pad(slow, ((0, 0), (0, NCHF - NCHS), (0, 0)),
                       constant_values=fill)
        return jnp.concatenate([fast, slow], axis=0)

    srcA = asym(edge_index[0], 0)
    dstA = asym(edge_index[1], DUMMY)
    bat3 = jnp.pad(batch, (0, NP - n),
                   constant_values=NG).reshape(GRID, 1, BR)

    zrows = jnp.zeros((RPT, HD), jnp.float32)
    ones_rows = jnp.ones((CH, HD), jnp.float32)

    l1b = lin1_b.reshape(1, HD)
    sgb = sage_b.reshape(4, 1, HD)
    clb = cell_b.reshape(2, 1, HD)
    rob = ro_b.reshape(1, HD)
    clsWp = jnp.pad(cls_W, ((0, 0), (0, HD - out_dim)))
    clsbp = jnp.pad(cls_b, (0, HD - out_dim)).reshape(1, HD)

    _agg = _get_agg()
    _deg = _get_deg()

    dp = _deg(dst3, zrows, ones_rows)
    h0 = _dense(xp, lin1_W, l1b)

    p = _agg(h0, srcA, dstA, zrows)
    h1, r1 = _combine(h0, p, dp, sage_Wl[0], sage_Wr[0], sgb[0])

    p = _agg(r1, srcA, dstA, zrows)
    h2, r2 = _combine(r1, p, dp, sage_Wl[1], sage_Wr[1], sgb[1])

    h3 = _dense(r2, cell_W[0], clb[0])

    p = _agg(h3, srcA, dstA, zrows)
    h4, r4 = _combine(h3, p, dp, sage_Wl[2], sage_Wr[2], sgb[2])

    p = _agg(r4, srcA, dstA, zrows)
    h5, r5 = _combine(r4, p, dp, sage_Wl[3], sage_Wr[3], sgb[3])

    h6 = _dense(r5, cell_W[1], clb[1])

    out = _readout(h6, bat3, ro_W, rob, clsWp, clsbp)
    return out[:, :out_dim]


# speedup vs baseline: 1.1343x; 1.1343x over previous
import functools

import jax
import jax.numpy as jnp
from jax import lax
from jax.experimental import pallas as pl
from jax.experimental.pallas import tpu as pltpu
from jax.experimental.pallas import tpu_sc as plsc

NP = 10240
HD = 128
NG = 64
CH = 128
NCH = 80
TILES = 32
EPAD = TILES * NCH * CH
DUMMY = 10000
RPT = NP // 16
BR = 1024
GRID = NP // BR

NCHF = 120
NCHS = 38
EDGF = 16 * NCHF * CH
EPAD_A = 16 * (NCHF + NCHS) * CH


def _agg_body(v_hbm, src_hbm, dst_hbm, z_hbm, out_hbm,
              acc, srci, dsti, rows, s0, s1):
    sems = (s0, s1)
    c = lax.axis_index("c")
    s = lax.axis_index("s")
    wid = c * 16 + s

    pltpu.sync_copy(z_hbm, acc.at[pl.ds(s * RPT, RPT)])
    plsc.subcore_barrier()

    def group(g, carry):
        pltpu.sync_copy(src_hbm.at[wid, pl.ds(2 * g, 2)], srci)
        pltpu.sync_copy(dst_hbm.at[wid, pl.ds(2 * g, 2)], dsti)
        handles = [pltpu.async_copy(v_hbm.at[srci.at[b]], rows.at[b], sems[b])
                   for b in range(2)]
        for b in range(2):
            handles[b].wait()
            pltpu.sync_copy(rows.at[b], acc.at[dsti.at[b]], add=True)
        return carry

    ngroups = lax.select(c == 0, NCHF // 2, NCHS // 2)
    lax.fori_loop(0, ngroups, group, 0)
    plsc.subcore_barrier()

    for i in range(RPT // CH):
        r0 = s * RPT + i * CH
        pltpu.sync_copy(acc.at[pl.ds(r0, CH)], rows.at[0])
        pltpu.sync_copy(rows.at[0], out_hbm.at[c, pl.ds(r0, CH)])


@functools.lru_cache(maxsize=None)
def _get_agg():
    return pl.kernel(
        _agg_body,
        out_type=jax.ShapeDtypeStruct((2, NP, HD), jnp.float32),
        mesh=plsc.VectorSubcoreMesh(core_axis_name="c", subcore_axis_name="s"),
        scratch_types=[
            pltpu.VMEM_SHARED((NP, HD), jnp.float32),
            pltpu.VMEM((2, CH), jnp.int32),
            pltpu.VMEM((2, CH), jnp.int32),
            pltpu.VMEM((2, CH, HD), jnp.float32),
            pltpu.SemaphoreType.DMA,
            pltpu.SemaphoreType.DMA,
        ],
    )


def _deg_body(dst_hbm, z_hbm, ones_hbm, dout_hbm, dacc, dsti, ones_v):
    c = lax.axis_index("c")
    s = lax.axis_index("s")
    wid = c * 16 + s

    pltpu.sync_copy(z_hbm, dacc.at[pl.ds(s * RPT, RPT)])
    pltpu.sync_copy(ones_hbm, ones_v)
    plsc.subcore_barrier()

    def group(g, carry):
        pltpu.sync_copy(dst_hbm.at[wid, pl.ds(2 * g, 2)], dsti)
        for b in range(2):
            pltpu.sync_copy(ones_v, dacc.at[dsti.at[b]], add=True)
        return carry

    lax.fori_loop(0, NCH // 2, group, 0)
    plsc.subcore_barrier()

    for i in range(RPT // CH):
        r0 = s * RPT + i * CH
        pltpu.sync_copy(dacc.at[pl.ds(r0, CH)], ones_v)
        pltpu.sync_copy(ones_v, dout_hbm.at[c, pl.ds(r0, CH)])


@functools.lru_cache(maxsize=None)
def _get_deg():
    return pl.kernel(
        _deg_body,
        out_type=jax.ShapeDtypeStruct((2, NP, HD), jnp.float32),
        mesh=plsc.VectorSubcoreMesh(core_axis_name="c", subcore_axis_name="s"),
        scratch_types=[
            pltpu.VMEM_SHARED((NP, HD), jnp.float32),
            pltpu.VMEM((2, CH), jnp.int32),
            pltpu.VMEM((CH, HD), jnp.float32),
        ],
    )


def _dense_body(x_ref, w_ref, b_ref, o_ref):
    y = jnp.dot(x_ref[...], w_ref[...], preferred_element_type=jnp.float32)
    o_ref[...] = jnp.maximum(y + b_ref[...], 0.0)


_dense = pl.pallas_call(
    _dense_body,
    grid=(GRID,),
    in_specs=[
        pl.BlockSpec((BR, HD), lambda i: (i, 0)),
        pl.BlockSpec((HD, HD), lambda i: (0, 0)),
        pl.BlockSpec((1, HD), lambda i: (0, 0)),
    ],
    out_specs=pl.BlockSpec((BR, HD), lambda i: (i, 0)),
    out_shape=jax.ShapeDtypeStruct((NP, HD), jnp.float32),
)


def _sage_step(t_ref, p_ref, d_ref, wl_ref, wr_ref, b_ref):
    t = t_ref[...]
    agg = p_ref[0] + p_ref[1]
    deg = d_ref[0, :, 0:1] + d_ref[1, :, 0:1]
    mean = agg / jnp.maximum(deg, 1.0)
    y = jnp.dot(t, wl_ref[...], preferred_element_type=jnp.float32)
    y = y + jnp.dot(mean, wr_ref[...], preferred_element_type=jnp.float32)
    h = jnp.maximum(y + b_ref[...], 0.0)
    return t + h


def _combine_body(t_ref, p_ref, d_ref, wl_ref, wr_ref, b_ref, s_ref):
    s_ref[...] = _sage_step(t_ref, p_ref, d_ref, wl_ref, wr_ref, b_ref)


def _combine_dense_body(t_ref, p_ref, d_ref, wl_ref, wr_ref, b_ref,
                        w2_ref, b2_ref, o_ref):
    r = _sage_step(t_ref, p_ref, d_ref, wl_ref, wr_ref, b_ref)
    y = jnp.dot(r, w2_ref[...], preferred_element_type=jnp.float32)
    o_ref[...] = jnp.maximum(y + b2_ref[...], 0.0)


_combine = pl.pallas_call(
    _combine_body,
    grid=(GRID,),
    in_specs=[
        pl.BlockSpec((BR, HD), lambda i: (i, 0)),
        pl.BlockSpec((2, BR, HD), lambda i: (0, i, 0)),
        pl.BlockSpec((2, BR, HD), lambda i: (0, i, 0)),
        pl.BlockSpec((HD, HD), lambda i: (0, 0)),
        pl.BlockSpec((HD, HD), lambda i: (0, 0)),
        pl.BlockSpec((1, HD), lambda i: (0, 0)),
    ],
    out_specs=pl.BlockSpec((BR, HD), lambda i: (i, 0)),
    out_shape=jax.ShapeDtypeStruct((NP, HD), jnp.float32),
)


_combine_dense = pl.pallas_call(
    _combine_dense_body,
    grid=(GRID,),
    in_specs=[
        pl.BlockSpec((BR, HD), lambda i: (i, 0)),
        pl.BlockSpec((2, BR, HD), lambda i: (0, i, 0)),
        pl.BlockSpec((2, BR, HD), lambda i: (0, i, 0)),
        pl.BlockSpec((HD, HD), lambda i: (0, 0)),
        pl.BlockSpec((HD, HD), lambda i: (0, 0)),
        pl.BlockSpec((1, HD), lambda i: (0, 0)),
        pl.BlockSpec((HD, HD), lambda i: (0, 0)),
        pl.BlockSpec((1, HD), lambda i: (0, 0)),
    ],
    out_specs=pl.BlockSpec((BR, HD), lambda i: (i, 0)),
    out_shape=jax.ShapeDtypeStruct((NP, HD), jnp.float32),
)


def _tail_body(t_ref, p_ref, d_ref, wl_ref, wr_ref, b_ref, w2_ref, b2_ref,
               bat_ref, row_ref, rob_ref, clw_ref, clb_ref,
               o_ref, accy, accc):
    i = pl.program_id(0)
    r = _sage_step(t_ref, p_ref, d_ref, wl_ref, wr_ref, b_ref)
    y = jnp.dot(r, w2_ref[...], preferred_element_type=jnp.float32)
    h = jnp.maximum(y + b2_ref[...], 0.0)
    bids = bat_ref[0, 0, :]
    onehot = (bids[:, None] ==
              lax.broadcasted_iota(jnp.int32, (BR, NG), 1)).astype(jnp.float32)
    dn = (((0,), (0,)), ((), ()))
    py = lax.dot_general(onehot, h, dn,
                         preferred_element_type=jnp.float32)
    pc = lax.dot_general(onehot, jnp.ones((BR, HD), jnp.float32), dn,
                         preferred_element_type=jnp.float32)

    @pl.when(i == 0)
    def _init():
        accy[...] = py
        accc[...] = pc

    @pl.when(i > 0)
    def _accum():
        accy[...] = accy[...] + py
        accc[...] = accc[...] + pc

    @pl.when(i == GRID - 1)
    def _final():
        pooled = accy[...] / jnp.maximum(accc[...], 1.0)
        r = jnp.maximum(
            jnp.dot(pooled, row_ref[...], preferred_element_type=jnp.float32)
            + rob_ref[...], 0.0)
        o_ref[...] = (jnp.dot(r, clw_ref[...],
                              preferred_element_type=jnp.float32)
                      + clb_ref[...])


_tail = pl.pallas_call(
    _tail_body,
    grid=(GRID,),
    in_specs=[
        pl.BlockSpec((BR, HD), lambda i: (i, 0)),
        pl.BlockSpec((2, BR, HD), lambda i: (0, i, 0)),
        pl.BlockSpec((2, BR, HD), lambda i: (0, i, 0)),
        pl.BlockSpec((HD, HD), lambda i: (0, 0)),
        pl.BlockSpec((HD, HD), lambda i: (0, 0)),
        pl.BlockSpec((1, HD), lambda i: (0, 0)),
        pl.BlockSpec((HD, HD), lambda i: (0, 0)),
        pl.BlockSpec((1, HD), lambda i: (0, 0)),
        pl.BlockSpec((1, 1, BR), lambda i: (i, 0, 0)),
        pl.BlockSpec((HD, HD), lambda i: (0, 0)),
        pl.BlockSpec((1, HD), lambda i: (0, 0)),
        pl.BlockSpec((HD, HD), lambda i: (0, 0)),
        pl.BlockSpec((1, HD), lambda i: (0, 0)),
    ],
    out_specs=pl.BlockSpec((NG, HD), lambda i: (0, 0)),
    out_shape=jax.ShapeDtypeStruct((NG, HD), jnp.float32),
    scratch_shapes=[
        pltpu.VMEM((NG, HD), jnp.float32),
        pltpu.VMEM((NG, HD), jnp.float32),
    ],
)


def kernel(x, edge_index, batch, lin1_W, lin1_b, sage_Wl, sage_Wr, sage_b,
           cell_W, cell_b, ro_W, ro_b, cls_W, cls_b):
    n, d = x.shape
    e = edge_index.shape[1]
    out_dim = cls_W.shape[1]

    xp = jnp.pad(x, ((0, NP - n), (0, 0)))
    dst3 = jnp.pad(edge_index[1], (0, EPAD - e),
                   constant_values=DUMMY).reshape(TILES, NCH, CH)

    def asym(idx, fill):
        fast = idx[:EDGF].reshape(16, NCHF, CH)
        slow = jnp.pad(idx[EDGF:], (0, EPAD_A - EDGF - (e - EDGF)),
                       constant_values=fill).reshape(16, NCHS, CH)
        slow = jnp.pad(slow, ((0, 0), (0, NCHF - NCHS), (0, 0)),
                       constant_values=fill)
        return jnp.concatenate([fast, slow], axis=0)

    srcA = asym(edge_index[0], 0)
    dstA = asym(edge_index[1], DUMMY)
    bat3 = jnp.pad(batch, (0, NP - n),
                   constant_values=NG).reshape(GRID, 1, BR)

    zrows = jnp.zeros((RPT, HD), jnp.float32)
    ones_rows = jnp.ones((CH, HD), jnp.float32)

    l1b = lin1_b.reshape(1, HD)
    sgb = sage_b.reshape(4, 1, HD)
    clb = cell_b.reshape(2, 1, HD)
    rob = ro_b.reshape(1, HD)
    clsWp = jnp.pad(cls_W, ((0, 0), (0, HD - out_dim)))
    clsbp = jnp.pad(cls_b, (0, HD - out_dim)).reshape(1, HD)

    _agg = _get_agg()
    _deg = _get_deg()

    dp = _deg(dst3, zrows, ones_rows)
    h0 = _dense(xp, lin1_W, l1b)

    p = _agg(h0, srcA, dstA, zrows)
    r1 = _combine(h0, p, dp, sage_Wl[0], sage_Wr[0], sgb[0])

    p = _agg(r1, srcA, dstA, zrows)
    h3 = _combine_dense(r1, p, dp, sage_Wl[1], sage_Wr[1], sgb[1],
                        cell_W[0], clb[0])

    p = _agg(h3, srcA, dstA, zrows)
    r4 = _combine(h3, p, dp, sage_Wl[2], sage_Wr[2], sgb[2])

    p = _agg(r4, srcA, dstA, zrows)
    out = _tail(r4, p, dp, sage_Wl[3], sage_Wr[3], sgb[3],
                cell_W[1], clb[1], bat3, ro_W, rob, clsWp, clsbp)
    return out[:, :out_dim]
